# baseline (speedup 1.0000x reference)
import functools

import jax
import jax.numpy as jnp
from jax import lax
from jax.experimental import pallas as pl
from jax.experimental.pallas import tpu as pltpu
from jax.experimental.pallas import tpu_sc as plsc

_B = 16384
_D = 64


def _sc_gather(idx, t_mlp, t_mf):
    info = plsc.get_sparse_core_info()
    nc, ns = info.num_cores, info.num_subcores
    nw = nc * ns
    bpw = _B // nw

    mesh = plsc.VectorSubcoreMesh(core_axis_name="c", subcore_axis_name="s")

    @functools.partial(
        pl.kernel,
        out_type=(
            jax.ShapeDtypeStruct((_B, _D), jnp.float32),
            jax.ShapeDtypeStruct((_B, _D), jnp.float32),
        ),
        mesh=mesh,
        compiler_params=pltpu.CompilerParams(use_tc_tiling_on_sc=False),
        scratch_types=[
            pltpu.VMEM((bpw,), jnp.int32),
            pltpu.VMEM((bpw, _D), jnp.float32),
            pltpu.VMEM((bpw, _D), jnp.float32),
            pltpu.SemaphoreType.DMA,
            pltpu.SemaphoreType.DMA,
        ],
    )
    def k(idx_hbm, t1_hbm, t2_hbm, o1_hbm, o2_hbm, idx_v, r1_v, r2_v, s1, s2):
        wid = lax.axis_index("s") * nc + lax.axis_index("c")
        base = wid * bpw
        pltpu.sync_copy(idx_hbm.at[pl.ds(base, bpw)], idx_v)
        c1 = pltpu.async_copy(t1_hbm.at[idx_v], r1_v, s1)
        c2 = pltpu.async_copy(t2_hbm.at[idx_v], r2_v, s2)
        c1.wait()
        pltpu.sync_copy(r1_v, o1_hbm.at[pl.ds(base, bpw)])
        c2.wait()
        pltpu.sync_copy(r2_v, o2_hbm.at[pl.ds(base, bpw)])

    return k(idx, t_mlp, t_mf)


def _tc_body(g1_ref, g2_ref, um_ref, uf_ref, w1_ref, b1_ref, w2_ref,
             b2_ref, w3_ref, b3_ref, wa_ref, ba_ref, o_ref):
    g1 = g1_ref[...]
    w1 = w1_ref[...]
    h1 = jnp.dot(g1, w1[_D:, :], preferred_element_type=jnp.float32)
    h1 = h1 + jnp.dot(um_ref[...], w1[:_D, :],
                      preferred_element_type=jnp.float32)
    h1 = jnp.maximum(h1 + b1_ref[...], 0.0)
    h2 = jnp.maximum(
        jnp.dot(h1, w2_ref[...], preferred_element_type=jnp.float32)
        + b2_ref[...], 0.0)
    h3 = jnp.maximum(
        jnp.dot(h2, w3_ref[...], preferred_element_type=jnp.float32)
        + b3_ref[...], 0.0)
    wa = wa_ref[...]
    s = jnp.dot(h3, wa[:16, :], preferred_element_type=jnp.float32)
    s = s + jnp.dot(g2_ref[...] * uf_ref[...], wa[16:, :],
                    preferred_element_type=jnp.float32)
    o_ref[...] = jax.nn.sigmoid(s + ba_ref[...])[:, 0]


def _tc_mlp(g_mlp, g_mf, u_mlp, u_mf, w1t, b1, w2t, b2, w3t, b3, wat, ba):
    blk = 2048
    grid = _B // blk
    fixed = lambda shape: pl.BlockSpec(shape, lambda i: (0,) * len(shape))
    return pl.pallas_call(
        _tc_body,
        grid=(grid,),
        in_specs=[
            pl.BlockSpec((blk, _D), lambda i: (i, 0)),
            pl.BlockSpec((blk, _D), lambda i: (i, 0)),
            fixed((1, _D)),
            fixed((1, _D)),
            fixed((2 * _D, _D)),
            fixed((1, _D)),
            fixed((_D, 32)),
            fixed((1, 32)),
            fixed((32, 16)),
            fixed((1, 16)),
            fixed((16 + _D, 1)),
            fixed((1, 1)),
        ],
        out_specs=pl.BlockSpec((blk,), lambda i: (i,)),
        out_shape=jax.ShapeDtypeStruct((_B,), jnp.float32),
    )(g_mlp, g_mf, u_mlp, u_mf, w1t, b1, w2t, b2, w3t, b3, wat, ba)


def kernel(item_indices, emb_user_mlp, emb_item_mlp, emb_user_mf,
           emb_item_mf, W1, b1, W2, b2, W3, b3, Wa, ba):
    idx = item_indices - 1
    g_mlp, g_mf = _sc_gather(idx, emb_item_mlp, emb_item_mf)
    return _tc_mlp(
        g_mlp, g_mf, emb_user_mlp, emb_user_mf,
        W1.T, b1.reshape(1, -1), W2.T, b2.reshape(1, -1),
        W3.T, b3.reshape(1, -1), Wa.T, ba.reshape(1, 1))
